# baseline (speedup 1.0000x reference)
import jax
import jax.numpy as jnp
from jax.experimental import pallas as pl
from jax.experimental.pallas import tpu as pltpu

_NBUF = 3


def _make_body(B, L, D, BLK):
    NSTEP = L // BLK

    def body(x_hbm, emb_hbm, o_hbm, xb, eb, sem_x, sem_o, sem_e):
        ecopy = pltpu.make_async_copy(emb_hbm, eb, sem_e)
        ecopy.start()

        def xcopy(s):
            slot = s % _NBUF
            return pltpu.make_async_copy(
                x_hbm.at[:, pl.ds(s * BLK, BLK), :], xb.at[slot], sem_x.at[slot]
            )

        def ocopy(s):
            slot = s % _NBUF
            return pltpu.make_async_copy(
                xb.at[slot], o_hbm.at[:, pl.ds(s * BLK, BLK), :], sem_o.at[slot]
            )

        for s in range(_NBUF - 1):
            if s < NSTEP:
                xcopy(s).start()
        for s in range(NSTEP):
            slot = s % _NBUF
            nxt = s + _NBUF - 1
            if nxt < NSTEP:
                if s >= 1:
                    ocopy(s - 1).wait()
                xcopy(nxt).start()
            xcopy(s).wait()
            if s == 0:
                ecopy.wait()
            for r in range(0, BLK, 8):
                ev = eb[pl.ds(s * BLK + r, 8), :]
                for b in range(B):
                    xb[slot, b, pl.ds(r, 8), :] = xb[slot, b, pl.ds(r, 8), :] + ev
            ocopy(s).start()
        for s in range(NSTEP - _NBUF, NSTEP):
            ocopy(s).wait()

    return body


def _pos_add_3d(x, emb_slice):
    B, L, D = x.shape
    BLK = 256
    return pl.pallas_call(
        _make_body(B, L, D, BLK),
        in_specs=[
            pl.BlockSpec(memory_space=pltpu.MemorySpace.HBM),
            pl.BlockSpec(memory_space=pltpu.MemorySpace.HBM),
        ],
        out_specs=pl.BlockSpec(memory_space=pltpu.MemorySpace.HBM),
        out_shape=jax.ShapeDtypeStruct((B, L, D), x.dtype),
        scratch_shapes=[
            pltpu.VMEM((_NBUF, B, BLK, D), x.dtype),
            pltpu.VMEM((L, D), x.dtype),
            pltpu.SemaphoreType.DMA((_NBUF,)),
            pltpu.SemaphoreType.DMA((_NBUF,)),
            pltpu.SemaphoreType.DMA,
        ],
    )(x, emb_slice)


def kernel(x, emb_table):
    if x.ndim == 3:
        L = x.shape[-2]
        return _pos_add_3d(x, emb_table[:L])
    b, h, l, d = x.shape
    xr = jnp.reshape(jnp.transpose(x, (0, 2, 1, 3)), (b, l, h * d))
    xr = _pos_add_3d(xr, emb_table[:l])
    return jnp.transpose(jnp.reshape(xr, (b, l, h, d)), (0, 2, 1, 3))
